# baseline (speedup 1.0000x reference)
import jax
import jax.numpy as jnp
from jax.experimental import pallas as pl
from jax.experimental.pallas import tpu as pltpu

_RANK = 4
_SCALING = 8 / 4
_EPS = 1e-5

_BR = 256
_CR = 8


def _ln_kernel(x_ref, sa_ref, sb_ref, ha_ref, hb_ref, o_ref):
    scale = jnp.sum(sa_ref[...] * sb_ref[...], axis=0, keepdims=True) * _SCALING
    shift = jnp.sum(ha_ref[...] * hb_ref[...], axis=0, keepdims=True) * _SCALING

    def chunk(c, _):
        base = c * _CR
        xs = x_ref[pl.ds(base, _CR), :]
        mean = jnp.mean(xs, axis=-1, keepdims=True)
        xc = xs - mean
        var = jnp.mean(xc * xc, axis=-1, keepdims=True)
        o_ref[pl.ds(base, _CR), :] = (
            xc * (jax.lax.rsqrt(var + _EPS) * scale) + shift
        )
        return ()

    jax.lax.fori_loop(0, _BR // _CR, chunk, ())


def kernel(x, lora_scale_A, lora_scale_B, lora_shift_A, lora_shift_B):
    B, S, N = x.shape
    rows = B * S
    x2 = x.reshape(rows, N)
    sa = lora_scale_A.T
    ha = lora_shift_A.T

    lora_spec = pl.BlockSpec((_RANK, N), lambda i: (0, 0))
    out = pl.pallas_call(
        _ln_kernel,
        grid=(rows // _BR,),
        in_specs=[
            pl.BlockSpec((_BR, N), lambda i: (i, 0)),
            lora_spec,
            lora_spec,
            lora_spec,
            lora_spec,
        ],
        out_specs=pl.BlockSpec((_BR, N), lambda i: (i, 0)),
        out_shape=jax.ShapeDtypeStruct((rows, N), x.dtype),
        compiler_params=pltpu.CompilerParams(
            dimension_semantics=("parallel",),
            vmem_limit_bytes=63 * 1024 * 1024,
        ),
    )(x2, sa, lora_scale_B, ha, lora_shift_B)
    return out.reshape(B, S, N)


# speedup vs baseline: 2.1929x; 2.1929x over previous
import jax
import jax.numpy as jnp
from jax.experimental import pallas as pl
from jax.experimental.pallas import tpu as pltpu

_RANK = 4
_SCALING = 8 / 4
_EPS = 1e-5

_BR = 256


def _ln_kernel(x_ref, sa_ref, sb_ref, ha_ref, hb_ref, o_ref):
    scale = jnp.sum(sa_ref[...] * sb_ref[...], axis=0, keepdims=True) * _SCALING
    shift = jnp.sum(ha_ref[...] * hb_ref[...], axis=0, keepdims=True) * _SCALING
    x = x_ref[...]
    mean = jnp.mean(x, axis=-1, keepdims=True)
    xc = x - mean
    var = jnp.mean(xc * xc, axis=-1, keepdims=True)
    o_ref[...] = xc * (jax.lax.rsqrt(var + _EPS) * scale) + shift


def kernel(x, lora_scale_A, lora_scale_B, lora_shift_A, lora_shift_B):
    B, S, N = x.shape
    rows = B * S
    x2 = x.reshape(rows, N)
    sa = lora_scale_A.T
    ha = lora_shift_A.T

    lora_spec = pl.BlockSpec((_RANK, N), lambda i: (0, 0))
    out = pl.pallas_call(
        _ln_kernel,
        grid=(rows // _BR,),
        in_specs=[
            pl.BlockSpec((_BR, N), lambda i: (i, 0)),
            lora_spec,
            lora_spec,
            lora_spec,
            lora_spec,
        ],
        out_specs=pl.BlockSpec((_BR, N), lambda i: (i, 0)),
        out_shape=jax.ShapeDtypeStruct((rows, N), x.dtype),
        compiler_params=pltpu.CompilerParams(
            dimension_semantics=("parallel",),
            vmem_limit_bytes=63 * 1024 * 1024,
        ),
    )(x2, sa, lora_scale_B, ha, lora_shift_B)
    return out.reshape(B, S, N)


# speedup vs baseline: 2.2120x; 1.0087x over previous
import jax
import jax.numpy as jnp
from jax.experimental import pallas as pl
from jax.experimental.pallas import tpu as pltpu

_RANK = 4
_SCALING = 8 / 4
_EPS = 1e-5

_N = 8192
_ROWS = 8192
_NCORES = 2
_BR = 128
_NBUF = 6
_STEPS = _ROWS // (_NCORES * _BR)


def _outer(x_hbm, sa_ref, sb_ref, ha_ref, hb_ref, o_hbm):
    core = pl.program_id(0)
    scale = jnp.sum(sa_ref[...] * sb_ref[...], axis=0, keepdims=True) * _SCALING
    shift = jnp.sum(ha_ref[...] * hb_ref[...], axis=0, keepdims=True) * _SCALING

    def body(x_ref, o_ref):
        x = x_ref[...]
        mean = jnp.mean(x, axis=-1, keepdims=True)
        xc = x - mean
        var = jnp.mean(xc * xc, axis=-1, keepdims=True)
        o_ref[...] = xc * (jax.lax.rsqrt(var + _EPS) * scale) + shift

    pipe = pltpu.emit_pipeline(
        body,
        grid=(_STEPS,),
        in_specs=[
            pl.BlockSpec(
                (_BR, _N),
                lambda j: (core * _STEPS + j, 0),
                pipeline_mode=pl.Buffered(buffer_count=_NBUF),
            )
        ],
        out_specs=[
            pl.BlockSpec(
                (_BR, _N),
                lambda j: (core * _STEPS + j, 0),
                pipeline_mode=pl.Buffered(buffer_count=2),
            )
        ],
    )
    pipe(x_hbm, o_hbm)


def kernel(x, lora_scale_A, lora_scale_B, lora_shift_A, lora_shift_B):
    B, S, N = x.shape
    rows = B * S
    x2 = x.reshape(rows, N)
    sa = lora_scale_A.T
    ha = lora_shift_A.T

    lora_spec = pl.BlockSpec((_RANK, N), lambda i: (0, 0))
    out = pl.pallas_call(
        _outer,
        grid=(_NCORES,),
        in_specs=[
            pl.BlockSpec(memory_space=pl.ANY),
            lora_spec,
            lora_spec,
            lora_spec,
            lora_spec,
        ],
        out_specs=pl.BlockSpec(memory_space=pl.ANY),
        out_shape=jax.ShapeDtypeStruct((rows, N), x.dtype),
        compiler_params=pltpu.CompilerParams(
            dimension_semantics=("parallel",),
            vmem_limit_bytes=63 * 1024 * 1024,
        ),
    )(x2, sa, lora_scale_B, ha, lora_shift_B)
    return out.reshape(B, S, N)
